# baseline (speedup 1.0000x reference)
import functools

import jax
import jax.numpy as jnp
from jax import lax
from jax.experimental import pallas as pl
from jax.experimental.pallas import tpu as pltpu
from jax.experimental.pallas import tpu_sc as plsc

BATCH = 16384
EMBED_DIM = 32
WIDE = 128
PACK = WIDE // EMBED_DIM
NUM_CORES = 2
NUM_SUBCORES = 16
NUM_WORKERS = NUM_CORES * NUM_SUBCORES
ROWS_PER_WORKER = BATCH // NUM_WORKERS
CHUNK = 128
NUM_CHUNKS = ROWS_PER_WORKER // CHUNK


def _gather_sc(uidx, iidx, ut_wide, it_wide):
    mesh = plsc.VectorSubcoreMesh(core_axis_name="c", subcore_axis_name="s")

    @functools.partial(
        pl.kernel,
        out_type=[
            jax.ShapeDtypeStruct((BATCH, WIDE), jnp.float32),
            jax.ShapeDtypeStruct((BATCH, WIDE), jnp.float32),
        ],
        mesh=mesh,
        scratch_types=[
            pltpu.VMEM((NUM_CHUNKS, CHUNK), jnp.int32),
            pltpu.VMEM((NUM_CHUNKS, CHUNK), jnp.int32),
            pltpu.VMEM((2, CHUNK, WIDE), jnp.float32),
            pltpu.VMEM((2, CHUNK, WIDE), jnp.float32),
            pltpu.SemaphoreType.DMA,
            pltpu.SemaphoreType.DMA,
        ],
    )
    def gather_kernel(uidx_hbm, iidx_hbm, ut_hbm, it_hbm, uw_hbm, iw_hbm,
                      uidx_v, iidx_v, ur_v, ir_v, gsem, osem):
        wid = lax.axis_index("s") * NUM_CORES + lax.axis_index("c")
        base = wid * ROWS_PER_WORKER
        pltpu.sync_copy(uidx_hbm.at[wid], uidx_v)
        pltpu.sync_copy(iidx_hbm.at[wid], iidx_v)

        gathers = [None, None]
        drains = [None, None]

        def fire(j, slot):
            gathers[slot] = (
                pltpu.async_copy(ut_hbm.at[uidx_v.at[j]], ur_v.at[slot], gsem),
                pltpu.async_copy(it_hbm.at[iidx_v.at[j]], ir_v.at[slot], gsem),
            )

        def drain(j, slot):
            for g in gathers[slot]:
                g.wait()
            row0 = base + j * CHUNK
            drains[slot] = (
                pltpu.async_copy(ur_v.at[slot], uw_hbm.at[pl.ds(row0, CHUNK)],
                                 osem),
                pltpu.async_copy(ir_v.at[slot], iw_hbm.at[pl.ds(row0, CHUNK)],
                                 osem),
            )

        for j in range(NUM_CHUNKS):
            slot = j % 2
            if drains[slot] is not None:
                for d in drains[slot]:
                    d.wait()
                drains[slot] = None
            fire(j, slot)
            if j >= 1:
                drain(j - 1, (j - 1) % 2)
        drain(NUM_CHUNKS - 1, (NUM_CHUNKS - 1) % 2)
        for slot in (0, 1):
            if drains[slot] is not None:
                for d in drains[slot]:
                    d.wait()

    return gather_kernel(uidx, iidx, ut_wide, it_wide)


def _mlp_body(uw_ref, iw_ref, uo_ref, io_ref, w1_ref, b1_ref, w2_ref, b2_ref,
              w3_ref, b3_ref, w4_ref, b4_ref, o_ref):
    uw = uw_ref[...]
    iw = iw_ref[...]
    uo = uo_ref[...]
    io = io_ref[...]
    ue = jnp.zeros(uw.shape[:1] + (EMBED_DIM,), jnp.float32)
    ie = ue
    for g in range(PACK):
        sl = slice(g * EMBED_DIM, (g + 1) * EMBED_DIM)
        ue = ue + uw[:, sl] * (uo == g).astype(jnp.float32)
        ie = ie + iw[:, sl] * (io == g).astype(jnp.float32)
    h = (jnp.dot(ue, w1_ref[0:EMBED_DIM, :], preferred_element_type=jnp.float32)
         + jnp.dot(ie, w1_ref[EMBED_DIM:2 * EMBED_DIM, :],
                   preferred_element_type=jnp.float32)
         + b1_ref[...])
    h = jnp.maximum(h, 0.0)
    h = jnp.maximum(jnp.dot(h, w2_ref[...], preferred_element_type=jnp.float32)
                    + b2_ref[...], 0.0)
    h = jnp.maximum(jnp.dot(h, w3_ref[...], preferred_element_type=jnp.float32)
                    + b3_ref[...], 0.0)
    y = jax.nn.sigmoid(jnp.dot(h, w4_ref[...], preferred_element_type=jnp.float32)
                       + b4_ref[...])
    o_ref[...] = y * 5.0 + 1.0


def _mlp_tc(uw, iw, uo, io, W1, b1, W2, b2, W3, b3, W4, b4):
    blk = 2048
    grid = (BATCH // blk,)
    full = lambda shape: pl.BlockSpec(shape, lambda i: (0,) * len(shape))
    return pl.pallas_call(
        _mlp_body,
        grid=grid,
        in_specs=[
            pl.BlockSpec((blk, WIDE), lambda i: (i, 0)),
            pl.BlockSpec((blk, WIDE), lambda i: (i, 0)),
            pl.BlockSpec((blk, 1), lambda i: (i, 0)),
            pl.BlockSpec((blk, 1), lambda i: (i, 0)),
            full(W1.shape), full(b1.shape),
            full(W2.shape), full(b2.shape),
            full(W3.shape), full(b3.shape),
            full(W4.shape), full(b4.shape),
        ],
        out_specs=pl.BlockSpec((blk, 1), lambda i: (i, 0)),
        out_shape=jax.ShapeDtypeStruct((BATCH, 1), jnp.float32),
    )(uw, iw, uo, io, W1, b1, W2, b2, W3, b3, W4, b4)


def kernel(user_indices, item_indices, emb_user, emb_item,
           W1, b1, W2, b2, W3, b3, W4, b4):
    ui = user_indices.astype(jnp.int32)
    ii = item_indices.astype(jnp.int32)
    uidx = (ui // PACK).reshape(NUM_WORKERS, NUM_CHUNKS, CHUNK)
    iidx = (ii // PACK).reshape(NUM_WORKERS, NUM_CHUNKS, CHUNK)
    ut_wide = emb_user.reshape(-1, WIDE)
    it_wide = emb_item.reshape(-1, WIDE)
    uw, iw = _gather_sc(uidx, iidx, ut_wide, it_wide)
    return _mlp_tc(uw, iw, (ui % PACK).reshape(BATCH, 1),
                   (ii % PACK).reshape(BATCH, 1),
                   W1, b1.reshape(1, -1), W2, b2.reshape(1, -1),
                   W3, b3.reshape(1, -1), W4, b4.reshape(1, -1))
